# Initial kernel scaffold; baseline (speedup 1.0000x reference)
import jax
import jax.numpy as jnp
from jax.experimental import pallas as pl


def kernel(x, adj, ln_g, ln_b, W1, b1, W2, b2, W3, b3):
    raise NotImplementedError("write your pallas kernel here")


# speedup vs baseline: 10.4788x; 10.4788x over previous
import functools

import jax
import jax.numpy as jnp
from jax import lax
from jax.experimental import pallas as pl
from jax.experimental.pallas import tpu as pltpu
from jax.experimental.pallas import tpu_sc as plsc

N = 10000
E = 320000
FEAT = 128
HID = 128
NCLASS = 64

NC = 2
NS = 16
NW = NC * NS
EP = E // NW
K = 80
CHUNKS = EP // K
RB = 624
CR = 208
DEGW = 16

_mesh = lambda: plsc.VectorSubcoreMesh(
    core_axis_name="c", subcore_axis_name="s", num_cores=NC, num_subcores=NS)


def _zero_acc(sid, zbuf, acc):
    D = zbuf.shape[1]

    def fill(i, c):
        for j in range(D // 16):
            zbuf[i, pl.ds(j * 16, 16)] = jnp.zeros((16,), jnp.float32)
        return c

    lax.fori_loop(0, CR, fill, 0)
    for k in range(RB // CR):
        pltpu.sync_copy(zbuf.at[:], acc.at[pl.ds(sid * RB + k * CR, CR)])

    @pl.when(sid == NS - 1)
    def _():
        pltpu.sync_copy(zbuf.at[pl.ds(0, 16)], acc.at[pl.ds(NS * RB, 16)])


def _flush_acc(sid, cid, acc, out_hbm):
    for k in range(RB // CR):
        r0 = sid * RB + k * CR
        pltpu.sync_copy(acc.at[pl.ds(r0, CR)], out_hbm.at[cid].at[pl.ds(r0, CR)])

    @pl.when(sid == NS - 1)
    def _():
        pltpu.sync_copy(acc.at[pl.ds(NS * RB, 16)], out_hbm.at[cid].at[pl.ds(NS * RB, 16)])


def _deg_body(dst_hbm, out_hbm, didx, ones, zbuf, acc, sem):
    cid = lax.axis_index("c")
    sid = lax.axis_index("s")
    wid = cid * NS + sid

    def fill1(i, c):
        ones[i, :] = jnp.ones((16,), jnp.float32)
        return c

    lax.fori_loop(0, K, fill1, 0)
    _zero_acc(sid, zbuf, acc)
    plsc.subcore_barrier()

    def body(c, carry):
        base = wid * EP + c * K
        pltpu.sync_copy(dst_hbm.at[pl.ds(base, K)], didx)
        pltpu.sync_copy(ones, acc.at[didx], add=True)
        return carry

    lax.fori_loop(0, CHUNKS, body, 0)
    plsc.subcore_barrier()
    _flush_acc(sid, cid, acc, out_hbm)


@functools.cache
def _deg_call():
    return functools.partial(
        pl.kernel,
        out_type=jax.ShapeDtypeStruct((NC, N, DEGW), jnp.float32),
        mesh=_mesh(),
        scratch_types=[
            pltpu.VMEM((K,), jnp.int32),
            pltpu.VMEM((K, DEGW), jnp.float32),
            pltpu.VMEM((CR, DEGW), jnp.float32),
            pltpu.VMEM_SHARED((N, DEGW), jnp.float32),
            pltpu.SemaphoreType.DMA,
        ],
    )(_deg_body)


def _prop_body(y_hbm, src_hbm, dst_hbm, out_hbm, sidx, didx, rows, zbuf, acc, sem):
    D = rows.shape[1]
    cid = lax.axis_index("c")
    sid = lax.axis_index("s")
    wid = cid * NS + sid

    _zero_acc(sid, zbuf, acc)
    plsc.subcore_barrier()

    def body(c, carry):
        base = wid * EP + c * K
        pltpu.sync_copy(src_hbm.at[pl.ds(base, K)], sidx)
        pltpu.sync_copy(dst_hbm.at[pl.ds(base, K)], didx)
        pltpu.async_copy(y_hbm.at[sidx], rows, sem).wait()
        pltpu.sync_copy(rows, acc.at[didx], add=True)
        return carry

    lax.fori_loop(0, CHUNKS, body, 0)
    plsc.subcore_barrier()
    _flush_acc(sid, cid, acc, out_hbm)


@functools.cache
def _make_prop(D):
    return functools.partial(
        pl.kernel,
        out_type=jax.ShapeDtypeStruct((NC, N, D), jnp.float32),
        mesh=_mesh(),
        scratch_types=[
            pltpu.VMEM((K,), jnp.int32),
            pltpu.VMEM((K,), jnp.int32),
            pltpu.VMEM((K, D), jnp.float32),
            pltpu.VMEM((CR, D), jnp.float32),
            pltpu.VMEM_SHARED((N, D), jnp.float32),
            pltpu.SemaphoreType.DMA,
        ],
    )(_prop_body)


BR = 1000
GRID = N // BR


def _pre_body(x_ref, g_ref, b_ref, w_ref, degp_ref, y_ref, dinv_ref):
    x = x_ref[...]
    m = jnp.mean(x, axis=1, keepdims=True)
    v = jnp.mean((x - m) ** 2, axis=1, keepdims=True)
    h = (x - m) * lax.rsqrt(v + 1e-5) * g_ref[...] + b_ref[...]
    deg = 1.0 + degp_ref[0, :, 0:1] + degp_ref[1, :, 0:1]
    dinv = lax.rsqrt(deg)
    xw = jnp.dot(h, w_ref[...], preferred_element_type=jnp.float32)
    y_ref[...] = dinv * xw
    dinv_ref[...] = dinv


def _pre(x, g, b, w, degp):
    return pl.pallas_call(
        _pre_body,
        grid=(GRID,),
        in_specs=[
            pl.BlockSpec((BR, FEAT), lambda i: (i, 0)),
            pl.BlockSpec((1, FEAT), lambda i: (0, 0)),
            pl.BlockSpec((1, FEAT), lambda i: (0, 0)),
            pl.BlockSpec((FEAT, HID), lambda i: (0, 0)),
            pl.BlockSpec((NC, BR, DEGW), lambda i: (0, i, 0)),
        ],
        out_specs=[
            pl.BlockSpec((BR, HID), lambda i: (i, 0)),
            pl.BlockSpec((BR, 1), lambda i: (i, 0)),
        ],
        out_shape=[
            jax.ShapeDtypeStruct((N, HID), jnp.float32),
            jax.ShapeDtypeStruct((N, 1), jnp.float32),
        ],
    )(x, g, b, w, degp)


def _mid_body(p_ref, y_ref, dinv_ref, b_ref, w_ref, o_ref):
    dinv = dinv_ref[...]
    s = p_ref[0] + p_ref[1] + y_ref[...]
    t = jnp.maximum(dinv * s + b_ref[...], 0.0)
    o_ref[...] = dinv * jnp.dot(t, w_ref[...], preferred_element_type=jnp.float32)


def _mid(p, y, dinv, b, w):
    Din, Dout = w.shape
    return pl.pallas_call(
        _mid_body,
        grid=(GRID,),
        in_specs=[
            pl.BlockSpec((NC, BR, Din), lambda i: (0, i, 0)),
            pl.BlockSpec((BR, Din), lambda i: (i, 0)),
            pl.BlockSpec((BR, 1), lambda i: (i, 0)),
            pl.BlockSpec((1, Din), lambda i: (0, 0)),
            pl.BlockSpec((Din, Dout), lambda i: (0, 0)),
        ],
        out_specs=pl.BlockSpec((BR, Dout), lambda i: (i, 0)),
        out_shape=jax.ShapeDtypeStruct((N, Dout), jnp.float32),
    )(p, y, dinv, b, w)


def _final_body(p_ref, y_ref, dinv_ref, b_ref, o_ref):
    s = p_ref[0] + p_ref[1] + y_ref[...]
    o = dinv_ref[...] * s[:, :NCLASS] + b_ref[...]
    o = o - jnp.max(o, axis=1, keepdims=True)
    o_ref[...] = o - jnp.log(jnp.sum(jnp.exp(o), axis=1, keepdims=True))


def _final(p, y, dinv, b):
    return pl.pallas_call(
        _final_body,
        grid=(GRID,),
        in_specs=[
            pl.BlockSpec((NC, BR, HID), lambda i: (0, i, 0)),
            pl.BlockSpec((BR, HID), lambda i: (i, 0)),
            pl.BlockSpec((BR, 1), lambda i: (i, 0)),
            pl.BlockSpec((1, NCLASS), lambda i: (0, 0)),
        ],
        out_specs=pl.BlockSpec((BR, NCLASS), lambda i: (i, 0)),
        out_shape=jax.ShapeDtypeStruct((N, NCLASS), jnp.float32),
    )(p, y, dinv, b)


@jax.jit
def kernel(x, adj, ln_g, ln_b, W1, b1, W2, b2, W3, b3):
    src = adj[0]
    dst = adj[1]
    degp = _deg_call()(dst)
    y1, dinv = _pre(x, ln_g.reshape(1, -1), ln_b.reshape(1, -1), W1, degp)
    p1 = _make_prop(HID)(y1, src, dst)
    y2 = _mid(p1, y1, dinv, b1.reshape(1, -1), W2)
    p2 = _make_prop(HID)(y2, src, dst)
    W3p = jnp.concatenate([W3, jnp.zeros((HID, HID - NCLASS), W3.dtype)], axis=1)
    y3 = _mid(p2, y2, dinv, b2.reshape(1, -1), W3p)
    p3 = _make_prop(HID)(y3, src, dst)
    return _final(p3, y3, dinv, b3.reshape(1, -1))


# speedup vs baseline: 15.4462x; 1.4740x over previous
import functools

import jax
import jax.numpy as jnp
from jax import lax
from jax.experimental import pallas as pl
from jax.experimental.pallas import tpu as pltpu
from jax.experimental.pallas import tpu_sc as plsc

N = 10000
E = 320000
FEAT = 128
HID = 128
NCLASS = 64

NC = 2
NS = 16
NW = NC * NS
EP = E // NW
K = 80
CHUNKS = EP // K
RB = 624
ZR = 48
DEGW = 16

_mesh = lambda: plsc.VectorSubcoreMesh(
    core_axis_name="c", subcore_axis_name="s", num_cores=NC, num_subcores=NS)


def _row_to_buf(srcref, c, dstref):
    for j in range(K // 16):
        dstref[pl.ds(j * 16, 16)] = srcref[pl.ds(c * K + j * 16, 16)]


def _zero_acc(sid, zbuf, acc):
    D = zbuf.shape[1]

    def fill(i, c):
        for j in range(D // 16):
            zbuf[i, pl.ds(j * 16, 16)] = jnp.zeros((16,), jnp.float32)
        return c

    lax.fori_loop(0, ZR, fill, 0)
    for k in range(RB // ZR):
        pltpu.sync_copy(zbuf.at[:], acc.at[pl.ds(sid * RB + k * ZR, ZR)])

    @pl.when(sid == NS - 1)
    def _():
        pltpu.sync_copy(zbuf.at[pl.ds(0, 16)], acc.at[pl.ds(NS * RB, 16)])


def _flush_acc(sid, cid, acc, out_hbm):
    pltpu.sync_copy(acc.at[pl.ds(sid * RB, RB)], out_hbm.at[cid].at[pl.ds(sid * RB, RB)])

    @pl.when(sid == NS - 1)
    def _():
        pltpu.sync_copy(acc.at[pl.ds(NS * RB, 16)], out_hbm.at[cid].at[pl.ds(NS * RB, 16)])


def _deg_body(dst_hbm, out_hbm, didx, dsc, ones, zbuf, acc, sem):
    cid = lax.axis_index("c")
    sid = lax.axis_index("s")
    wid = cid * NS + sid

    cp = pltpu.make_async_copy(dst_hbm.at[wid], didx, sem)
    cp.start()

    def fill1(i, c):
        ones[i, :] = jnp.ones((16,), jnp.float32)
        return c

    lax.fori_loop(0, K, fill1, 0)
    _zero_acc(sid, zbuf, acc)
    cp.wait()
    plsc.subcore_barrier()

    def body(c, carry):
        _row_to_buf(didx, c, dsc)
        pltpu.sync_copy(ones, acc.at[dsc], add=True)
        return carry

    lax.fori_loop(0, CHUNKS, body, 0)
    plsc.subcore_barrier()
    _flush_acc(sid, cid, acc, out_hbm)


@functools.cache
def _deg_call():
    return functools.partial(
        pl.kernel,
        out_type=jax.ShapeDtypeStruct((NC, N, DEGW), jnp.float32),
        mesh=_mesh(),
        scratch_types=[
            pltpu.VMEM((EP,), jnp.int32),
            pltpu.VMEM((K,), jnp.int32),
            pltpu.VMEM((K, DEGW), jnp.float32),
            pltpu.VMEM((ZR, DEGW), jnp.float32),
            pltpu.VMEM_SHARED((N, DEGW), jnp.float32),
            pltpu.SemaphoreType.DMA,
        ],
    )(_deg_body)


def _prop_body(y_hbm, src_hbm, dst_hbm, out_hbm, sidx, didx, ssc0, ssc1, dsc,
               rows0, rows1, zbuf, acc, sem0, sem1):
    cid = lax.axis_index("c")
    sid = lax.axis_index("s")
    wid = cid * NS + sid

    cps = pltpu.make_async_copy(src_hbm.at[wid], sidx, sem0)
    cpd = pltpu.make_async_copy(dst_hbm.at[wid], didx, sem1)
    cps.start()
    cpd.start()
    _zero_acc(sid, zbuf, acc)
    cps.wait()
    cpd.wait()
    plsc.subcore_barrier()

    def body(c, carry):
        _row_to_buf(sidx, c, ssc0)
        _row_to_buf(didx, c, dsc)
        pltpu.async_copy(y_hbm.at[ssc0], rows0, sem0).wait()
        pltpu.sync_copy(rows0, acc.at[dsc], add=True)
        return carry

    lax.fori_loop(0, CHUNKS, body, 0)
    plsc.subcore_barrier()
    _flush_acc(sid, cid, acc, out_hbm)


@functools.cache
def _make_prop(D):
    assert CHUNKS % 2 == 1
    return functools.partial(
        pl.kernel,
        out_type=jax.ShapeDtypeStruct((NC, N, D), jnp.float32),
        mesh=_mesh(),
        scratch_types=[
            pltpu.VMEM((EP,), jnp.int32),
            pltpu.VMEM((EP,), jnp.int32),
            pltpu.VMEM((K,), jnp.int32),
            pltpu.VMEM((K,), jnp.int32),
            pltpu.VMEM((K,), jnp.int32),
            pltpu.VMEM((K, D), jnp.float32),
            pltpu.VMEM((K, D), jnp.float32),
            pltpu.VMEM((ZR, D), jnp.float32),
            pltpu.VMEM_SHARED((N, D), jnp.float32),
            pltpu.SemaphoreType.DMA,
            pltpu.SemaphoreType.DMA,
        ],
    )(_prop_body)


BR = 1000
GRID = N // BR


def _pre_body(x_ref, g_ref, b_ref, w_ref, degp_ref, y_ref, dinv_ref):
    x = x_ref[...]
    m = jnp.mean(x, axis=1, keepdims=True)
    v = jnp.mean((x - m) ** 2, axis=1, keepdims=True)
    h = (x - m) * lax.rsqrt(v + 1e-5) * g_ref[...] + b_ref[...]
    deg = 1.0 + degp_ref[0, :, 0:1] + degp_ref[1, :, 0:1]
    dinv = lax.rsqrt(deg)
    xw = jnp.dot(h, w_ref[...], preferred_element_type=jnp.float32)
    y_ref[...] = dinv * xw
    dinv_ref[...] = dinv


def _pre(x, g, b, w, degp):
    return pl.pallas_call(
        _pre_body,
        grid=(GRID,),
        in_specs=[
            pl.BlockSpec((BR, FEAT), lambda i: (i, 0)),
            pl.BlockSpec((1, FEAT), lambda i: (0, 0)),
            pl.BlockSpec((1, FEAT), lambda i: (0, 0)),
            pl.BlockSpec((FEAT, HID), lambda i: (0, 0)),
            pl.BlockSpec((NC, BR, DEGW), lambda i: (0, i, 0)),
        ],
        out_specs=[
            pl.BlockSpec((BR, HID), lambda i: (i, 0)),
            pl.BlockSpec((BR, 1), lambda i: (i, 0)),
        ],
        out_shape=[
            jax.ShapeDtypeStruct((N, HID), jnp.float32),
            jax.ShapeDtypeStruct((N, 1), jnp.float32),
        ],
    )(x, g, b, w, degp)


def _mid_body(p_ref, y_ref, dinv_ref, b_ref, w_ref, o_ref):
    dinv = dinv_ref[...]
    s = p_ref[0] + p_ref[1] + y_ref[...]
    t = jnp.maximum(dinv * s + b_ref[...], 0.0)
    o_ref[...] = dinv * jnp.dot(t, w_ref[...], preferred_element_type=jnp.float32)


def _mid(p, y, dinv, b, w):
    Din, Dout = w.shape
    return pl.pallas_call(
        _mid_body,
        grid=(GRID,),
        in_specs=[
            pl.BlockSpec((NC, BR, Din), lambda i: (0, i, 0)),
            pl.BlockSpec((BR, Din), lambda i: (i, 0)),
            pl.BlockSpec((BR, 1), lambda i: (i, 0)),
            pl.BlockSpec((1, Din), lambda i: (0, 0)),
            pl.BlockSpec((Din, Dout), lambda i: (0, 0)),
        ],
        out_specs=pl.BlockSpec((BR, Dout), lambda i: (i, 0)),
        out_shape=jax.ShapeDtypeStruct((N, Dout), jnp.float32),
    )(p, y, dinv, b, w)


def _final_body(p_ref, y_ref, dinv_ref, b_ref, o_ref):
    s = p_ref[0] + p_ref[1] + y_ref[...]
    o = dinv_ref[...] * s[:, :NCLASS] + b_ref[...]
    o = o - jnp.max(o, axis=1, keepdims=True)
    o_ref[...] = o - jnp.log(jnp.sum(jnp.exp(o), axis=1, keepdims=True))


def _final(p, y, dinv, b):
    return pl.pallas_call(
        _final_body,
        grid=(GRID,),
        in_specs=[
            pl.BlockSpec((NC, BR, HID), lambda i: (0, i, 0)),
            pl.BlockSpec((BR, HID), lambda i: (i, 0)),
            pl.BlockSpec((BR, 1), lambda i: (i, 0)),
            pl.BlockSpec((1, NCLASS), lambda i: (0, 0)),
        ],
        out_specs=pl.BlockSpec((BR, NCLASS), lambda i: (i, 0)),
        out_shape=jax.ShapeDtypeStruct((N, NCLASS), jnp.float32),
    )(p, y, dinv, b)


@jax.jit
def kernel(x, adj, ln_g, ln_b, W1, b1, W2, b2, W3, b3):
    src = adj[0].reshape(NW, EP)
    dst = adj[1].reshape(NW, EP)
    degp = _deg_call()(dst)
    y1, dinv = _pre(x, ln_g.reshape(1, -1), ln_b.reshape(1, -1), W1, degp)
    p1 = _make_prop(HID)(y1, src, dst)
    y2 = _mid(p1, y1, dinv, b1.reshape(1, -1), W2)
    p2 = _make_prop(HID)(y2, src, dst)
    W3p = jnp.concatenate([W3, jnp.zeros((HID, HID - NCLASS), W3.dtype)], axis=1)
    y3 = _mid(p2, y2, dinv, b2.reshape(1, -1), W3p)
    p3 = _make_prop(HID)(y3, src, dst)
    return _final(p3, y3, dinv, b3.reshape(1, -1))


# speedup vs baseline: 18.8570x; 1.2208x over previous
import functools

import jax
import jax.numpy as jnp
from jax import lax
from jax.experimental import pallas as pl
from jax.experimental.pallas import tpu as pltpu
from jax.experimental.pallas import tpu_sc as plsc

N = 10000
E = 320000
FEAT = 128
HID = 128
NCLASS = 64

NC = 2
NS = 16
NW = NC * NS
EP = E // NW
K = 80
CHUNKS = EP // K
RB = 624
ZR = 48
DEGW = 16

_mesh = lambda: plsc.VectorSubcoreMesh(
    core_axis_name="c", subcore_axis_name="s", num_cores=NC, num_subcores=NS)


def _row_to_buf(srcref, c, dstref):
    for j in range(K // 16):
        dstref[pl.ds(j * 16, 16)] = srcref[pl.ds(c * K + j * 16, 16)]


def _zero_acc(sid, zbuf, acc):
    D = zbuf.shape[1]

    def fill(i, c):
        for j in range(D // 16):
            zbuf[i, pl.ds(j * 16, 16)] = jnp.zeros((16,), jnp.float32)
        return c

    lax.fori_loop(0, ZR, fill, 0)
    for k in range(RB // ZR):
        pltpu.sync_copy(zbuf.at[:], acc.at[pl.ds(sid * RB + k * ZR, ZR)])

    @pl.when(sid == NS - 1)
    def _():
        pltpu.sync_copy(zbuf.at[pl.ds(0, 16)], acc.at[pl.ds(NS * RB, 16)])


def _flush_acc(sid, cid, acc, out_hbm):
    pltpu.sync_copy(acc.at[pl.ds(sid * RB, RB)], out_hbm.at[cid].at[pl.ds(sid * RB, RB)])

    @pl.when(sid == NS - 1)
    def _():
        pltpu.sync_copy(acc.at[pl.ds(NS * RB, 16)], out_hbm.at[cid].at[pl.ds(NS * RB, 16)])


def _deg_body(dst_hbm, out_hbm, didx, dsc, ones, zbuf, acc, sem):
    cid = lax.axis_index("c")
    sid = lax.axis_index("s")
    wid = cid * NS + sid

    cp = pltpu.make_async_copy(dst_hbm.at[wid], didx, sem)
    cp.start()

    def fill1(i, c):
        ones[i, :] = jnp.ones((16,), jnp.float32)
        return c

    lax.fori_loop(0, K, fill1, 0)
    _zero_acc(sid, zbuf, acc)
    cp.wait()
    plsc.subcore_barrier()

    def body(c, carry):
        _row_to_buf(didx, c, dsc)
        pltpu.sync_copy(ones, acc.at[dsc], add=True)
        return carry

    lax.fori_loop(0, CHUNKS, body, 0)
    plsc.subcore_barrier()
    _flush_acc(sid, cid, acc, out_hbm)


@functools.cache
def _deg_call():
    return functools.partial(
        pl.kernel,
        out_type=jax.ShapeDtypeStruct((NC, N, DEGW), jnp.float32),
        mesh=_mesh(),
        scratch_types=[
            pltpu.VMEM((EP,), jnp.int32),
            pltpu.VMEM((K,), jnp.int32),
            pltpu.VMEM((K, DEGW), jnp.float32),
            pltpu.VMEM((ZR, DEGW), jnp.float32),
            pltpu.VMEM_SHARED((N, DEGW), jnp.float32),
            pltpu.SemaphoreType.DMA,
        ],
    )(_deg_body)


def _prop_body(y_hbm, src_hbm, dst_hbm, out_hbm, sidx, didx, ssc0, ssc1, dsc,
               rows0, rows1, zbuf, acc, sem0, sem1):
    cid = lax.axis_index("c")
    sid = lax.axis_index("s")
    wid = cid * NS + sid

    cps = pltpu.make_async_copy(src_hbm.at[wid], sidx, sem0)
    cpd = pltpu.make_async_copy(dst_hbm.at[wid], didx, sem1)
    cps.start()
    cpd.start()
    _zero_acc(sid, zbuf, acc)
    cps.wait()
    cpd.wait()
    plsc.subcore_barrier()

    def g_start(c, isc, buf, sem):
        _row_to_buf(sidx, c, isc)
        pltpu.make_async_copy(y_hbm.at[isc], buf, sem).start()

    def g_wait(isc, buf, sem):
        pltpu.make_async_copy(y_hbm.at[isc], buf, sem).wait()

    def s_add(c, buf):
        _row_to_buf(didx, c, dsc)
        pltpu.sync_copy(buf, acc.at[dsc], add=True)

    def body(i, carry):
        c0 = 2 * i
        g_start(c0, ssc0, rows0, sem0)
        g_start(c0 + 1, ssc1, rows1, sem1)
        g_wait(ssc0, rows0, sem0)
        g_wait(ssc1, rows1, sem1)
        s_add(c0, rows0)
        s_add(c0 + 1, rows1)
        return carry

    lax.fori_loop(0, CHUNKS // 2, body, 0)
    g_start(CHUNKS - 1, ssc0, rows0, sem0)
    g_wait(ssc0, rows0, sem0)
    s_add(CHUNKS - 1, rows0)
    plsc.subcore_barrier()
    _flush_acc(sid, cid, acc, out_hbm)


@functools.cache
def _make_prop(D):
    assert CHUNKS % 2 == 1
    return functools.partial(
        pl.kernel,
        out_type=jax.ShapeDtypeStruct((NC, N, D), jnp.float32),
        mesh=_mesh(),
        scratch_types=[
            pltpu.VMEM((EP,), jnp.int32),
            pltpu.VMEM((EP,), jnp.int32),
            pltpu.VMEM((K,), jnp.int32),
            pltpu.VMEM((K,), jnp.int32),
            pltpu.VMEM((K,), jnp.int32),
            pltpu.VMEM((K, D), jnp.float32),
            pltpu.VMEM((K, D), jnp.float32),
            pltpu.VMEM((ZR, D), jnp.float32),
            pltpu.VMEM_SHARED((N, D), jnp.float32),
            pltpu.SemaphoreType.DMA,
            pltpu.SemaphoreType.DMA,
        ],
    )(_prop_body)


BR = 1000
GRID = N // BR


def _pre_body(x_ref, g_ref, b_ref, w_ref, degp_ref, y_ref, dinv_ref):
    x = x_ref[...]
    m = jnp.mean(x, axis=1, keepdims=True)
    v = jnp.mean((x - m) ** 2, axis=1, keepdims=True)
    h = (x - m) * lax.rsqrt(v + 1e-5) * g_ref[...] + b_ref[...]
    deg = 1.0 + degp_ref[0, :, 0:1] + degp_ref[1, :, 0:1]
    dinv = lax.rsqrt(deg)
    xw = jnp.dot(h, w_ref[...], preferred_element_type=jnp.float32)
    y_ref[...] = dinv * xw
    dinv_ref[...] = dinv


def _pre(x, g, b, w, degp):
    return pl.pallas_call(
        _pre_body,
        grid=(GRID,),
        in_specs=[
            pl.BlockSpec((BR, FEAT), lambda i: (i, 0)),
            pl.BlockSpec((1, FEAT), lambda i: (0, 0)),
            pl.BlockSpec((1, FEAT), lambda i: (0, 0)),
            pl.BlockSpec((FEAT, HID), lambda i: (0, 0)),
            pl.BlockSpec((NC, BR, DEGW), lambda i: (0, i, 0)),
        ],
        out_specs=[
            pl.BlockSpec((BR, HID), lambda i: (i, 0)),
            pl.BlockSpec((BR, 1), lambda i: (i, 0)),
        ],
        out_shape=[
            jax.ShapeDtypeStruct((N, HID), jnp.float32),
            jax.ShapeDtypeStruct((N, 1), jnp.float32),
        ],
    )(x, g, b, w, degp)


def _mid_body(p_ref, y_ref, dinv_ref, b_ref, w_ref, o_ref):
    dinv = dinv_ref[...]
    s = p_ref[0] + p_ref[1] + y_ref[...]
    t = jnp.maximum(dinv * s + b_ref[...], 0.0)
    o_ref[...] = dinv * jnp.dot(t, w_ref[...], preferred_element_type=jnp.float32)


def _mid(p, y, dinv, b, w):
    Din, Dout = w.shape
    return pl.pallas_call(
        _mid_body,
        grid=(GRID,),
        in_specs=[
            pl.BlockSpec((NC, BR, Din), lambda i: (0, i, 0)),
            pl.BlockSpec((BR, Din), lambda i: (i, 0)),
            pl.BlockSpec((BR, 1), lambda i: (i, 0)),
            pl.BlockSpec((1, Din), lambda i: (0, 0)),
            pl.BlockSpec((Din, Dout), lambda i: (0, 0)),
        ],
        out_specs=pl.BlockSpec((BR, Dout), lambda i: (i, 0)),
        out_shape=jax.ShapeDtypeStruct((N, Dout), jnp.float32),
    )(p, y, dinv, b, w)


def _final_body(p_ref, y_ref, dinv_ref, b_ref, o_ref):
    s = p_ref[0] + p_ref[1] + y_ref[...]
    o = dinv_ref[...] * s[:, :NCLASS] + b_ref[...]
    o = o - jnp.max(o, axis=1, keepdims=True)
    o_ref[...] = o - jnp.log(jnp.sum(jnp.exp(o), axis=1, keepdims=True))


def _final(p, y, dinv, b):
    return pl.pallas_call(
        _final_body,
        grid=(GRID,),
        in_specs=[
            pl.BlockSpec((NC, BR, HID), lambda i: (0, i, 0)),
            pl.BlockSpec((BR, HID), lambda i: (i, 0)),
            pl.BlockSpec((BR, 1), lambda i: (i, 0)),
            pl.BlockSpec((1, NCLASS), lambda i: (0, 0)),
        ],
        out_specs=pl.BlockSpec((BR, NCLASS), lambda i: (i, 0)),
        out_shape=jax.ShapeDtypeStruct((N, NCLASS), jnp.float32),
    )(p, y, dinv, b)


@jax.jit
def kernel(x, adj, ln_g, ln_b, W1, b1, W2, b2, W3, b3):
    src = adj[0].reshape(NW, EP)
    dst = adj[1].reshape(NW, EP)
    degp = _deg_call()(dst)
    y1, dinv = _pre(x, ln_g.reshape(1, -1), ln_b.reshape(1, -1), W1, degp)
    p1 = _make_prop(HID)(y1, src, dst)
    y2 = _mid(p1, y1, dinv, b1.reshape(1, -1), W2)
    p2 = _make_prop(HID)(y2, src, dst)
    W3p = jnp.concatenate([W3, jnp.zeros((HID, HID - NCLASS), W3.dtype)], axis=1)
    y3 = _mid(p2, y2, dinv, b2.reshape(1, -1), W3p)
    p3 = _make_prop(HID)(y3, src, dst)
    return _final(p3, y3, dinv, b3.reshape(1, -1))


# speedup vs baseline: 19.3211x; 1.0246x over previous
import functools

import jax
import jax.numpy as jnp
from jax import lax
from jax.experimental import pallas as pl
from jax.experimental.pallas import tpu as pltpu
from jax.experimental.pallas import tpu_sc as plsc

N = 10000
E = 320000
FEAT = 128
HID = 128
NCLASS = 64

NC = 2
NS = 16
NW = NC * NS
EP = E // NW
K = 80
CHUNKS = EP // K
RB = 624
ZR = 48
DEGW = 16

_mesh = lambda: plsc.VectorSubcoreMesh(
    core_axis_name="c", subcore_axis_name="s", num_cores=NC, num_subcores=NS)


def _row_to_buf(srcref, c, dstref):
    for j in range(K // 16):
        dstref[pl.ds(j * 16, 16)] = srcref[pl.ds(c * K + j * 16, 16)]


def _zero_acc(sid, zbuf, acc):
    D = zbuf.shape[1]

    def fill(i, c):
        for j in range(D // 16):
            zbuf[i, pl.ds(j * 16, 16)] = jnp.zeros((16,), jnp.float32)
        return c

    lax.fori_loop(0, ZR, fill, 0)
    for k in range(RB // ZR):
        pltpu.sync_copy(zbuf.at[:], acc.at[pl.ds(sid * RB + k * ZR, ZR)])

    @pl.when(sid == NS - 1)
    def _():
        pltpu.sync_copy(zbuf.at[pl.ds(0, 16)], acc.at[pl.ds(NS * RB, 16)])


def _flush_acc(sid, cid, acc, out_hbm):
    pltpu.sync_copy(acc.at[pl.ds(sid * RB, RB)], out_hbm.at[cid].at[pl.ds(sid * RB, RB)])

    @pl.when(sid == NS - 1)
    def _():
        pltpu.sync_copy(acc.at[pl.ds(NS * RB, 16)], out_hbm.at[cid].at[pl.ds(NS * RB, 16)])


def _deg_body(dst_hbm, out_hbm, didx, dsc, ones, zbuf, acc, sem):
    cid = lax.axis_index("c")
    sid = lax.axis_index("s")
    wid = cid * NS + sid

    cp = pltpu.make_async_copy(dst_hbm.at[wid], didx, sem)
    cp.start()

    def fill1(i, c):
        ones[i, :] = jnp.ones((16,), jnp.float32)
        return c

    lax.fori_loop(0, K, fill1, 0)
    _zero_acc(sid, zbuf, acc)
    cp.wait()
    plsc.subcore_barrier()

    def body(c, carry):
        _row_to_buf(didx, c, dsc)
        pltpu.sync_copy(ones, acc.at[dsc], add=True)
        return carry

    lax.fori_loop(0, CHUNKS, body, 0)
    plsc.subcore_barrier()
    _flush_acc(sid, cid, acc, out_hbm)


@functools.cache
def _deg_call():
    return functools.partial(
        pl.kernel,
        out_type=jax.ShapeDtypeStruct((NC, N, DEGW), jnp.float32),
        mesh=_mesh(),
        scratch_types=[
            pltpu.VMEM((EP,), jnp.int32),
            pltpu.VMEM((K,), jnp.int32),
            pltpu.VMEM((K, DEGW), jnp.float32),
            pltpu.VMEM((ZR, DEGW), jnp.float32),
            pltpu.VMEM_SHARED((N, DEGW), jnp.float32),
            pltpu.SemaphoreType.DMA,
        ],
    )(_deg_body)


def _prop_body(y_hbm, src_hbm, dst_hbm, out_hbm, sidx, didx, ssc0, ssc1,
               dsc0, dsc1, rows0, rows1, zbuf, acc, sem0, sem1, sem2, sem3):
    cid = lax.axis_index("c")
    sid = lax.axis_index("s")
    wid = cid * NS + sid

    cps = pltpu.make_async_copy(src_hbm.at[wid], sidx, sem0)
    cpd = pltpu.make_async_copy(dst_hbm.at[wid], didx, sem1)
    cps.start()
    cpd.start()
    _zero_acc(sid, zbuf, acc)
    cps.wait()
    cpd.wait()
    plsc.subcore_barrier()

    def g_start(c, isc, buf, sem):
        _row_to_buf(sidx, c, isc)
        pltpu.make_async_copy(y_hbm.at[isc], buf, sem).start()

    def g_wait(isc, buf, sem):
        pltpu.make_async_copy(y_hbm.at[isc], buf, sem).wait()

    def s_start(c, idsc, buf, sem):
        _row_to_buf(didx, c, idsc)
        pltpu.make_async_copy(buf, acc.at[idsc], sem).start(add=True)

    def s_wait(idsc, buf, sem):
        pltpu.make_async_copy(buf, acc.at[idsc], sem).wait()

    def body(i, carry):
        c0 = 2 * i
        g_start(c0, ssc0, rows0, sem0)
        g_start(c0 + 1, ssc1, rows1, sem1)
        g_wait(ssc0, rows0, sem0)
        g_wait(ssc1, rows1, sem1)
        s_start(c0, dsc0, rows0, sem2)
        s_start(c0 + 1, dsc1, rows1, sem3)
        s_wait(dsc0, rows0, sem2)
        s_wait(dsc1, rows1, sem3)
        return carry

    lax.fori_loop(0, CHUNKS // 2, body, 0)
    g_start(CHUNKS - 1, ssc0, rows0, sem0)
    g_wait(ssc0, rows0, sem0)
    s_start(CHUNKS - 1, dsc0, rows0, sem2)
    s_wait(dsc0, rows0, sem2)
    plsc.subcore_barrier()
    _flush_acc(sid, cid, acc, out_hbm)


@functools.cache
def _make_prop(D):
    assert CHUNKS % 2 == 1
    return functools.partial(
        pl.kernel,
        out_type=jax.ShapeDtypeStruct((NC, N, D), jnp.float32),
        mesh=_mesh(),
        scratch_types=[
            pltpu.VMEM((EP,), jnp.int32),
            pltpu.VMEM((EP,), jnp.int32),
            pltpu.VMEM((K,), jnp.int32),
            pltpu.VMEM((K,), jnp.int32),
            pltpu.VMEM((K,), jnp.int32),
            pltpu.VMEM((K,), jnp.int32),
            pltpu.VMEM((K, D), jnp.float32),
            pltpu.VMEM((K, D), jnp.float32),
            pltpu.VMEM((ZR, D), jnp.float32),
            pltpu.VMEM_SHARED((N, D), jnp.float32),
            pltpu.SemaphoreType.DMA,
            pltpu.SemaphoreType.DMA,
            pltpu.SemaphoreType.DMA,
            pltpu.SemaphoreType.DMA,
        ],
    )(_prop_body)


BR = 1000
GRID = N // BR


def _pre_body(x_ref, g_ref, b_ref, w_ref, degp_ref, y_ref, dinv_ref):
    x = x_ref[...]
    m = jnp.mean(x, axis=1, keepdims=True)
    v = jnp.mean((x - m) ** 2, axis=1, keepdims=True)
    h = (x - m) * lax.rsqrt(v + 1e-5) * g_ref[...] + b_ref[...]
    deg = 1.0 + degp_ref[0, :, 0:1] + degp_ref[1, :, 0:1]
    dinv = lax.rsqrt(deg)
    xw = jnp.dot(h, w_ref[...], preferred_element_type=jnp.float32)
    y_ref[...] = dinv * xw
    dinv_ref[...] = dinv


def _pre(x, g, b, w, degp):
    return pl.pallas_call(
        _pre_body,
        grid=(GRID,),
        in_specs=[
            pl.BlockSpec((BR, FEAT), lambda i: (i, 0)),
            pl.BlockSpec((1, FEAT), lambda i: (0, 0)),
            pl.BlockSpec((1, FEAT), lambda i: (0, 0)),
            pl.BlockSpec((FEAT, HID), lambda i: (0, 0)),
            pl.BlockSpec((NC, BR, DEGW), lambda i: (0, i, 0)),
        ],
        out_specs=[
            pl.BlockSpec((BR, HID), lambda i: (i, 0)),
            pl.BlockSpec((BR, 1), lambda i: (i, 0)),
        ],
        out_shape=[
            jax.ShapeDtypeStruct((N, HID), jnp.float32),
            jax.ShapeDtypeStruct((N, 1), jnp.float32),
        ],
    )(x, g, b, w, degp)


def _mid_body(p_ref, y_ref, dinv_ref, b_ref, w_ref, o_ref):
    dinv = dinv_ref[...]
    s = p_ref[0] + p_ref[1] + y_ref[...]
    t = jnp.maximum(dinv * s + b_ref[...], 0.0)
    o_ref[...] = dinv * jnp.dot(t, w_ref[...], preferred_element_type=jnp.float32)


def _mid(p, y, dinv, b, w):
    Din, Dout = w.shape
    return pl.pallas_call(
        _mid_body,
        grid=(GRID,),
        in_specs=[
            pl.BlockSpec((NC, BR, Din), lambda i: (0, i, 0)),
            pl.BlockSpec((BR, Din), lambda i: (i, 0)),
            pl.BlockSpec((BR, 1), lambda i: (i, 0)),
            pl.BlockSpec((1, Din), lambda i: (0, 0)),
            pl.BlockSpec((Din, Dout), lambda i: (0, 0)),
        ],
        out_specs=pl.BlockSpec((BR, Dout), lambda i: (i, 0)),
        out_shape=jax.ShapeDtypeStruct((N, Dout), jnp.float32),
    )(p, y, dinv, b, w)


def _final_body(p_ref, y_ref, dinv_ref, b_ref, o_ref):
    s = p_ref[0] + p_ref[1] + y_ref[...]
    o = dinv_ref[...] * s[:, :NCLASS] + b_ref[...]
    o = o - jnp.max(o, axis=1, keepdims=True)
    o_ref[...] = o - jnp.log(jnp.sum(jnp.exp(o), axis=1, keepdims=True))


def _final(p, y, dinv, b):
    return pl.pallas_call(
        _final_body,
        grid=(GRID,),
        in_specs=[
            pl.BlockSpec((NC, BR, HID), lambda i: (0, i, 0)),
            pl.BlockSpec((BR, HID), lambda i: (i, 0)),
            pl.BlockSpec((BR, 1), lambda i: (i, 0)),
            pl.BlockSpec((1, NCLASS), lambda i: (0, 0)),
        ],
        out_specs=pl.BlockSpec((BR, NCLASS), lambda i: (i, 0)),
        out_shape=jax.ShapeDtypeStruct((N, NCLASS), jnp.float32),
    )(p, y, dinv, b)


@jax.jit
def kernel(x, adj, ln_g, ln_b, W1, b1, W2, b2, W3, b3):
    src = adj[0].reshape(NW, EP)
    dst = adj[1].reshape(NW, EP)
    degp = _deg_call()(dst)
    y1, dinv = _pre(x, ln_g.reshape(1, -1), ln_b.reshape(1, -1), W1, degp)
    p1 = _make_prop(HID)(y1, src, dst)
    y2 = _mid(p1, y1, dinv, b1.reshape(1, -1), W2)
    p2 = _make_prop(HID)(y2, src, dst)
    W3p = jnp.concatenate([W3, jnp.zeros((HID, HID - NCLASS), W3.dtype)], axis=1)
    y3 = _mid(p2, y2, dinv, b2.reshape(1, -1), W3p)
    p3 = _make_prop(HID)(y3, src, dst)
    return _final(p3, y3, dinv, b3.reshape(1, -1))
